# baseline (speedup 1.0000x reference)
import functools

import jax
import jax.numpy as jnp
from jax import lax
from jax.experimental import pallas as pl
from jax.experimental.pallas import tpu as pltpu
from jax.experimental.pallas import tpu_sc as plsc

NCORES = 2
NSUB = 16
NW = NCORES * NSUB
CK = 128
NBUF = 4
DJ = 8
NWIN = 9
TEXT_PAD = (2 * NBUF + 4) * CK
OFF_PAD = 160


def _build_pooled(B, D):
    bags_per_w = B // NW
    totb = bags_per_w + 1
    mesh = plsc.VectorSubcoreMesh(
        core_axis_name="c", subcore_axis_name="s",
        num_cores=NCORES, num_subcores=NSUB)

    off_len = bags_per_w + OFF_PAD

    @functools.partial(
        pl.kernel,
        out_type=jax.ShapeDtypeStruct((B, D), jnp.float32),
        mesh=mesh,
        scratch_types=[
            pltpu.VMEM((off_len,), jnp.int32),
            pltpu.VMEM((2 * NBUF * CK,), jnp.int32),
            pltpu.VMEM((NBUF, CK, D), jnp.float32),
            pltpu.VMEM((bags_per_w, D), jnp.float32),
            [pltpu.SemaphoreType.DMA] * NBUF,
        ],
    )
    def pooled_kernel(text_hbm, off_hbm, table_hbm, out_hbm,
                      off_v, idx_v, rows_v, out_v, sems):
        wid = lax.axis_index("s") * NCORES + lax.axis_index("c")
        base_bag = pl.multiple_of(wid * bags_per_w, 8)
        pltpu.sync_copy(off_hbm.at[pl.ds(base_bag, off_len)], off_v)

        S = off_v[pl.ds(0, 16)][0]
        E = off_v[pl.ds(bags_per_w, 16)][0]
        SA = pl.multiple_of(S & jnp.int32(~(CK - 1)), 8)
        ngrp = jnp.maximum((E - SA + NBUF * CK) // (NBUF * CK),
                           jnp.int32(1))

        def fetch_idx(g, p):
            t0 = pl.multiple_of(SA + g * (NBUF * CK), 8)
            po = pl.multiple_of(p * (NBUF * CK), 8)
            pltpu.sync_copy(text_hbm.at[pl.ds(t0, NBUF * CK)],
                            idx_v.at[pl.ds(po, NBUF * CK)])

        def start_gather(s, p):
            po = pl.multiple_of(p * (NBUF * CK) + s * CK, 8)
            pltpu.make_async_copy(
                table_hbm.at[idx_v.at[pl.ds(po, CK)]],
                rows_v.at[s], sems[s]).start()

        def wait_gather(s, p):
            po = pl.multiple_of(p * (NBUF * CK) + s * CK, 8)
            pltpu.make_async_copy(
                table_hbm.at[idx_v.at[pl.ds(po, CK)]],
                rows_v.at[s], sems[s]).wait()

        def row_accum(s, lo, hi, accs):
            lo16 = jnp.minimum((lo + 15) & ~jnp.int32(15), hi)
            hi16 = jnp.maximum(hi & ~jnp.int32(15), lo16)

            def row_body(r, accs):
                return tuple(
                    accs[j] + rows_v[s, r, pl.ds(16 * j, 16)]
                    for j in range(DJ))

            def blk_body(k, accs):
                r0 = lo16 + k * 16
                for rr in range(16):
                    r = r0 + rr
                    accs = tuple(
                        accs[j] + rows_v[s, r, pl.ds(16 * j, 16)]
                        for j in range(DJ))
                return accs

            accs = lax.fori_loop(lo, lo16, row_body, accs)
            accs = lax.fori_loop(0, (hi16 - lo16) // 16, blk_body, accs)
            return lax.fori_loop(hi16, hi, row_body, accs)

        def process_chunk(c, s, carry):
            bq, accs = carry
            t0 = SA + c * CK
            t_end = t0 + CK
            ones = jnp.ones((16,), jnp.int32)
            zeros = jnp.zeros((16,), jnp.int32)
            cnt_v = zeros
            for w in range(NWIN):
                wv = off_v[pl.ds(bq + 16 * w, 16)]
                cnt_v = cnt_v + jnp.where(wv < t_end, ones, zeros)
            nb = jnp.int32(0)
            for i in range(16):
                nb = nb + cnt_v[i]
            nb = jnp.minimum(nb, totb - bq)

            def bound_body(k, car):
                bqk, pp, accs = car
                pos = off_v[pl.ds(bqk, 16)][0] - t0
                accs = row_accum(s, pp, pos, accs)
                bag = jnp.maximum(bqk - 1, 0)
                for j in range(DJ):
                    out_v[bag, pl.ds(16 * j, 16)] = accs[j]
                z = tuple(jnp.zeros((16,), jnp.float32) for _ in range(DJ))
                return (bqk + 1, pos, z)

            bq, pp, accs = lax.fori_loop(
                0, nb, bound_body, (bq, jnp.int32(0), accs))
            accs = row_accum(s, pp, jnp.int32(CK), accs)
            return (bq, accs)

        fetch_idx(jnp.int32(0), 0)
        for s in range(NBUF):
            start_gather(s, 0)

        def group_body(g, carry):
            p = g & 1
            pn = 1 - p
            fetch_idx(g + 1, pn)
            for s in range(NBUF):
                wait_gather(s, p)
                carry = process_chunk(g * NBUF + s, s, carry)
                start_gather(s, pn)
            return carry

        accs0 = tuple(jnp.zeros((16,), jnp.float32) for _ in range(DJ))
        carry = lax.fori_loop(0, ngrp, group_body, (jnp.int32(0), accs0))

        p_drain = ngrp & 1
        for s in range(NBUF):
            wait_gather(s, p_drain)

        def scale_body(g, _):
            g16 = pl.multiple_of(g * 16, 8)
            lo = off_v[pl.ds(g16, 16)]
            hi = off_v[pl.ds(g16 + 1, 16)]
            cv = jnp.maximum((hi - lo).astype(jnp.float32), jnp.float32(1.0))
            rv = lax.bitcast_convert_type(
                jnp.int32(0x7EF311C3) - lax.bitcast_convert_type(
                    cv, jnp.int32),
                jnp.float32)
            for _ in range(3):
                rv = rv * (jnp.float32(2.0) - cv * rv)
            for jj in range(16):
                b = g16 + jj
                sv = lax.broadcast_in_dim(rv[jj], (16,), ())
                for j in range(DJ):
                    out_v[b, pl.ds(16 * j, 16)] = (
                        out_v[b, pl.ds(16 * j, 16)] * sv)
            return _
        lax.fori_loop(0, bags_per_w // 16, scale_body, 0)

        pltpu.sync_copy(out_v, out_hbm.at[pl.ds(base_bag, bags_per_w)])

    return pooled_kernel


def _mlp_body(x_ref, w1_ref, b1_ref, w2_ref, b2_ref, w3_ref, b3_ref, o_ref):
    x = x_ref[...]
    h = jnp.maximum(
        jnp.dot(x, w1_ref[...], preferred_element_type=jnp.float32)
        + b1_ref[...], 0.0)
    h = jnp.maximum(
        jnp.dot(h, w2_ref[...], preferred_element_type=jnp.float32)
        + b2_ref[...], 0.0)
    o_ref[...] = (
        jnp.dot(h, w3_ref[...], preferred_element_type=jnp.float32)
        + b3_ref[...])


def _mlp(pooled, W1, b1, W2, b2, W3, b3):
    B, D = pooled.shape
    H1 = W1.shape[0]
    NC = W3.shape[0]
    BLK = 512
    grid = (B // BLK,)
    out = pl.pallas_call(
        _mlp_body,
        grid=grid,
        in_specs=[
            pl.BlockSpec((BLK, D), lambda i: (i, 0)),
            pl.BlockSpec((D, H1), lambda i: (0, 0)),
            pl.BlockSpec((1, H1), lambda i: (0, 0)),
            pl.BlockSpec((H1, D), lambda i: (0, 0)),
            pl.BlockSpec((1, D), lambda i: (0, 0)),
            pl.BlockSpec((D, NC), lambda i: (0, 0)),
            pl.BlockSpec((1, NC), lambda i: (0, 0)),
        ],
        out_specs=pl.BlockSpec((BLK, NC), lambda i: (i, 0)),
        out_shape=jax.ShapeDtypeStruct((B, NC), jnp.float32),
    )(pooled, W1.T, b1[None, :], W2.T, b2[None, :], W3.T, b3[None, :])
    return out


def kernel(text, offsets, table, W1, b1, W2, b2, W3, b3):
    N = text.shape[0]
    B = offsets.shape[0]
    off_ext = jnp.concatenate([offsets, jnp.full((OFF_PAD,), N, jnp.int32)])
    text_pad = jnp.concatenate([text, jnp.zeros((TEXT_PAD,), jnp.int32)])
    pooled = _build_pooled(B, table.shape[1])(text_pad, off_ext, table)
    return _mlp(pooled, W1, b1, W2, b2, W3, b3)


# speedup vs baseline: 1.7100x; 1.7100x over previous
import functools

import jax
import jax.numpy as jnp
from jax import lax
from jax.experimental import pallas as pl
from jax.experimental.pallas import tpu as pltpu
from jax.experimental.pallas import tpu_sc as plsc

NCORES = 2
NSUB = 16
NW = NCORES * NSUB
CK = 128
NBUF = 6
DJ = 8
NWIN = 9
TEXT_PAD = (2 * NBUF + 4) * CK
OFF_PAD = 160


def _build_pooled(B, D):
    bags_per_w = B // NW
    totb = bags_per_w + 1
    mesh = plsc.VectorSubcoreMesh(
        core_axis_name="c", subcore_axis_name="s",
        num_cores=NCORES, num_subcores=NSUB)

    off_len = bags_per_w + OFF_PAD

    @functools.partial(
        pl.kernel,
        out_type=jax.ShapeDtypeStruct((B, D), jnp.float32),
        mesh=mesh,
        scratch_types=[
            pltpu.VMEM((off_len,), jnp.int32),
            pltpu.VMEM((2 * NBUF * CK,), jnp.int32),
            pltpu.VMEM((NBUF, CK, D), jnp.float32),
            pltpu.VMEM((bags_per_w, D), jnp.float32),
            [pltpu.SemaphoreType.DMA] * NBUF,
            pltpu.SemaphoreType.DMA,
        ],
    )
    def pooled_kernel(text_hbm, off_hbm, table_hbm, out_hbm,
                      off_v, idx_v, rows_v, out_v, sems, fsem):
        wid = lax.axis_index("s") * NCORES + lax.axis_index("c")
        base_bag = pl.multiple_of(wid * bags_per_w, 8)
        pltpu.sync_copy(off_hbm.at[pl.ds(base_bag, off_len)], off_v)

        S = off_v[pl.ds(0, 16)][0]
        E = off_v[pl.ds(bags_per_w, 16)][0]
        SA = pl.multiple_of(S & jnp.int32(~(CK - 1)), 8)
        ngrp = jnp.maximum((E - SA + NBUF * CK) // (NBUF * CK),
                           jnp.int32(1))

        def fetch_idx_start(g, p):
            t0 = pl.multiple_of(SA + g * (NBUF * CK), 8)
            po = pl.multiple_of(p * (NBUF * CK), 8)
            pltpu.make_async_copy(
                text_hbm.at[pl.ds(t0, NBUF * CK)],
                idx_v.at[pl.ds(po, NBUF * CK)], fsem).start()

        def fetch_idx_wait(g, p):
            t0 = pl.multiple_of(SA + g * (NBUF * CK), 8)
            po = pl.multiple_of(p * (NBUF * CK), 8)
            pltpu.make_async_copy(
                text_hbm.at[pl.ds(t0, NBUF * CK)],
                idx_v.at[pl.ds(po, NBUF * CK)], fsem).wait()

        def start_gather(s, p):
            po = pl.multiple_of(p * (NBUF * CK) + s * CK, 8)
            pltpu.make_async_copy(
                table_hbm.at[idx_v.at[pl.ds(po, CK)]],
                rows_v.at[s], sems[s]).start()

        def wait_gather(s, p):
            po = pl.multiple_of(p * (NBUF * CK) + s * CK, 8)
            pltpu.make_async_copy(
                table_hbm.at[idx_v.at[pl.ds(po, CK)]],
                rows_v.at[s], sems[s]).wait()

        def row_accum(s, lo, hi, accs):
            def row_body(r, accs):
                return tuple(
                    accs[j] + rows_v[s, r, pl.ds(16 * j, 16)]
                    for j in range(DJ))
            return lax.fori_loop(lo, hi, row_body, accs)

        def process_chunk(c, s, carry):
            bq, accs = carry
            t0 = SA + c * CK
            t_end = t0 + CK
            ones = jnp.ones((16,), jnp.int32)
            zeros = jnp.zeros((16,), jnp.int32)
            cnt_v = zeros
            for w in range(NWIN):
                wv = off_v[pl.ds(bq + 16 * w, 16)]
                cnt_v = cnt_v + jnp.where(wv < t_end, ones, zeros)
            nb = jnp.int32(0)
            for i in range(16):
                nb = nb + cnt_v[i]
            nb = jnp.minimum(nb, totb - bq)

            def bound_body(k, car):
                bqk, pp, accs = car
                pos = off_v[pl.ds(bqk, 16)][0] - t0
                accs = row_accum(s, pp, pos, accs)
                bag = jnp.maximum(bqk - 1, 0)
                for j in range(DJ):
                    out_v[bag, pl.ds(16 * j, 16)] = accs[j]
                z = tuple(jnp.zeros((16,), jnp.float32) for _ in range(DJ))
                return (bqk + 1, pos, z)

            bq, pp, accs = lax.fori_loop(
                0, nb, bound_body, (bq, jnp.int32(0), accs))
            accs = row_accum(s, pp, jnp.int32(CK), accs)
            return (bq, accs)

        fetch_idx_start(jnp.int32(0), 0)
        fetch_idx_wait(jnp.int32(0), 0)
        for s in range(NBUF):
            start_gather(s, 0)

        def group_body(g, carry):
            p = g & 1
            pn = 1 - p
            fetch_idx_start(g + 1, pn)
            fetch_idx_wait(g + 1, pn)
            for s in range(NBUF):
                wait_gather(s, p)
                carry = process_chunk(g * NBUF + s, s, carry)
                start_gather(s, pn)
            return carry

        accs0 = tuple(jnp.zeros((16,), jnp.float32) for _ in range(DJ))
        carry = lax.fori_loop(0, ngrp, group_body, (jnp.int32(0), accs0))

        p_drain = ngrp & 1
        for s in range(NBUF):
            wait_gather(s, p_drain)

        def scale_body(g, _):
            g16 = pl.multiple_of(g * 16, 8)
            lo = off_v[pl.ds(g16, 16)]
            hi = off_v[pl.ds(g16 + 1, 16)]
            cv = jnp.maximum((hi - lo).astype(jnp.float32), jnp.float32(1.0))
            rv = lax.bitcast_convert_type(
                jnp.int32(0x7EF311C3) - lax.bitcast_convert_type(
                    cv, jnp.int32),
                jnp.float32)
            for _ in range(3):
                rv = rv * (jnp.float32(2.0) - cv * rv)
            for jj in range(16):
                b = g16 + jj
                sv = lax.broadcast_in_dim(rv[jj], (16,), ())
                for j in range(DJ):
                    out_v[b, pl.ds(16 * j, 16)] = (
                        out_v[b, pl.ds(16 * j, 16)] * sv)
            return _
        lax.fori_loop(0, bags_per_w // 16, scale_body, 0)

        pltpu.sync_copy(out_v, out_hbm.at[pl.ds(base_bag, bags_per_w)])

    return pooled_kernel


def _mlp_body(x_ref, w1_ref, b1_ref, w2_ref, b2_ref, w3_ref, b3_ref, o_ref):
    x = x_ref[...]
    h = jnp.maximum(
        jnp.dot(x, w1_ref[...], preferred_element_type=jnp.float32)
        + b1_ref[...], 0.0)
    h = jnp.maximum(
        jnp.dot(h, w2_ref[...], preferred_element_type=jnp.float32)
        + b2_ref[...], 0.0)
    o_ref[...] = (
        jnp.dot(h, w3_ref[...], preferred_element_type=jnp.float32)
        + b3_ref[...])


def _mlp(pooled, W1, b1, W2, b2, W3, b3):
    B, D = pooled.shape
    H1 = W1.shape[0]
    NC = W3.shape[0]
    BLK = 512
    grid = (B // BLK,)
    out = pl.pallas_call(
        _mlp_body,
        grid=grid,
        in_specs=[
            pl.BlockSpec((BLK, D), lambda i: (i, 0)),
            pl.BlockSpec((D, H1), lambda i: (0, 0)),
            pl.BlockSpec((1, H1), lambda i: (0, 0)),
            pl.BlockSpec((H1, D), lambda i: (0, 0)),
            pl.BlockSpec((1, D), lambda i: (0, 0)),
            pl.BlockSpec((D, NC), lambda i: (0, 0)),
            pl.BlockSpec((1, NC), lambda i: (0, 0)),
        ],
        out_specs=pl.BlockSpec((BLK, NC), lambda i: (i, 0)),
        out_shape=jax.ShapeDtypeStruct((B, NC), jnp.float32),
    )(pooled, W1.T, b1[None, :], W2.T, b2[None, :], W3.T, b3[None, :])
    return out


def kernel(text, offsets, table, W1, b1, W2, b2, W3, b3):
    N = text.shape[0]
    B = offsets.shape[0]
    off_ext = jnp.concatenate([offsets, jnp.full((OFF_PAD,), N, jnp.int32)])
    text_pad = jnp.concatenate([text, jnp.zeros((TEXT_PAD,), jnp.int32)])
    pooled = _build_pooled(B, table.shape[1])(text_pad, off_ext, table)
    return _mlp(pooled, W1, b1, W2, b2, W3, b3)


# speedup vs baseline: 1.8570x; 1.0860x over previous
import functools

import jax
import jax.numpy as jnp
from jax import lax
from jax.experimental import pallas as pl
from jax.experimental.pallas import tpu as pltpu
from jax.experimental.pallas import tpu_sc as plsc

NCORES = 2
NSUB = 16
NW = NCORES * NSUB
CK = 128
NBUF = 4
DJ = 8
NWIN = 9
TEXT_PAD = (2 * NBUF + 4) * CK
OFF_PAD = 160


def _build_pooled(B, D):
    bags_per_w = B // NW
    totb = bags_per_w + 1
    mesh = plsc.VectorSubcoreMesh(
        core_axis_name="c", subcore_axis_name="s",
        num_cores=NCORES, num_subcores=NSUB)

    off_len = bags_per_w + OFF_PAD

    @functools.partial(
        pl.kernel,
        out_type=jax.ShapeDtypeStruct((B, D), jnp.float32),
        mesh=mesh,
        scratch_types=[
            pltpu.VMEM((off_len,), jnp.int32),
            pltpu.VMEM((2 * NBUF * CK,), jnp.int32),
            pltpu.VMEM((NBUF, CK, D), jnp.float32),
            pltpu.VMEM((bags_per_w, D), jnp.float32),
            [pltpu.SemaphoreType.DMA] * NBUF,
            pltpu.SemaphoreType.DMA,
        ],
    )
    def pooled_kernel(text_hbm, off_hbm, table_hbm, out_hbm,
                      off_v, idx_v, rows_v, out_v, sems, fsem):
        wid = lax.axis_index("s") * NCORES + lax.axis_index("c")
        base_bag = pl.multiple_of(wid * bags_per_w, 8)
        pltpu.sync_copy(off_hbm.at[pl.ds(base_bag, off_len)], off_v)

        S = off_v[pl.ds(0, 16)][0]
        E = off_v[pl.ds(bags_per_w, 16)][0]
        SA = pl.multiple_of(S & jnp.int32(~(CK - 1)), 8)
        ngrp = jnp.maximum((E - SA + NBUF * CK) // (NBUF * CK),
                           jnp.int32(1))

        def fetch_idx_start(g, p):
            t0 = pl.multiple_of(SA + g * (NBUF * CK), 8)
            po = pl.multiple_of(p * (NBUF * CK), 8)
            pltpu.make_async_copy(
                text_hbm.at[pl.ds(t0, NBUF * CK)],
                idx_v.at[pl.ds(po, NBUF * CK)], fsem).start()

        def fetch_idx_wait(g, p):
            t0 = pl.multiple_of(SA + g * (NBUF * CK), 8)
            po = pl.multiple_of(p * (NBUF * CK), 8)
            pltpu.make_async_copy(
                text_hbm.at[pl.ds(t0, NBUF * CK)],
                idx_v.at[pl.ds(po, NBUF * CK)], fsem).wait()

        def start_gather(s, p):
            po = pl.multiple_of(p * (NBUF * CK) + s * CK, 8)
            pltpu.make_async_copy(
                table_hbm.at[idx_v.at[pl.ds(po, CK)]],
                rows_v.at[s], sems[s]).start()

        def wait_gather(s, p):
            po = pl.multiple_of(p * (NBUF * CK) + s * CK, 8)
            pltpu.make_async_copy(
                table_hbm.at[idx_v.at[pl.ds(po, CK)]],
                rows_v.at[s], sems[s]).wait()

        def row_accum(s, lo, hi, accs):
            def row_body(r, accs):
                return tuple(
                    accs[j] + rows_v[s, r, pl.ds(16 * j, 16)]
                    for j in range(DJ))
            return lax.fori_loop(lo, hi, row_body, accs)

        def process_chunk(c, s, carry):
            bq, accs = carry
            t0 = SA + c * CK
            t_end = t0 + CK
            ones = jnp.ones((16,), jnp.int32)
            zeros = jnp.zeros((16,), jnp.int32)
            cnt_v = zeros
            for w in range(NWIN):
                wv = off_v[pl.ds(bq + 16 * w, 16)]
                cnt_v = cnt_v + jnp.where(wv < t_end, ones, zeros)
            nb = jnp.int32(0)
            for i in range(16):
                nb = nb + cnt_v[i]
            nb = jnp.minimum(nb, totb - bq)

            def bound_body(k, car):
                bqk, pp, accs = car
                pos = off_v[pl.ds(bqk, 16)][0] - t0
                accs = row_accum(s, pp, pos, accs)
                bag = jnp.maximum(bqk - 1, 0)
                for j in range(DJ):
                    out_v[bag, pl.ds(16 * j, 16)] = accs[j]
                z = tuple(jnp.zeros((16,), jnp.float32) for _ in range(DJ))
                return (bqk + 1, pos, z)

            bq, pp, accs = lax.fori_loop(
                0, nb, bound_body, (bq, jnp.int32(0), accs))
            accs = row_accum(s, pp, jnp.int32(CK), accs)
            return (bq, accs)

        fetch_idx_start(jnp.int32(0), 0)
        fetch_idx_wait(jnp.int32(0), 0)
        for s in range(NBUF):
            start_gather(s, 0)

        def group_body(g, carry):
            p = g & 1
            pn = 1 - p
            fetch_idx_start(g + 1, pn)
            fetch_idx_wait(g + 1, pn)
            for s in range(NBUF):
                wait_gather(s, p)
                carry = process_chunk(g * NBUF + s, s, carry)
                start_gather(s, pn)
            return carry

        accs0 = tuple(jnp.zeros((16,), jnp.float32) for _ in range(DJ))
        carry = lax.fori_loop(0, ngrp, group_body, (jnp.int32(0), accs0))

        p_drain = ngrp & 1
        for s in range(NBUF):
            wait_gather(s, p_drain)

        def scale_body(g, _):
            g16 = pl.multiple_of(g * 16, 8)
            lo = off_v[pl.ds(g16, 16)]
            hi = off_v[pl.ds(g16 + 1, 16)]
            cv = jnp.maximum((hi - lo).astype(jnp.float32), jnp.float32(1.0))
            rv = lax.bitcast_convert_type(
                jnp.int32(0x7EF311C3) - lax.bitcast_convert_type(
                    cv, jnp.int32),
                jnp.float32)
            for _ in range(3):
                rv = rv * (jnp.float32(2.0) - cv * rv)
            for jj in range(16):
                b = g16 + jj
                sv = lax.broadcast_in_dim(rv[jj], (16,), ())
                for j in range(DJ):
                    out_v[b, pl.ds(16 * j, 16)] = (
                        out_v[b, pl.ds(16 * j, 16)] * sv)
            return _
        lax.fori_loop(0, bags_per_w // 16, scale_body, 0)

        pltpu.sync_copy(out_v, out_hbm.at[pl.ds(base_bag, bags_per_w)])

    return pooled_kernel


def _mlp_body(x_ref, w1_ref, b1_ref, w2_ref, b2_ref, w3_ref, b3_ref, o_ref):
    x = x_ref[...]
    h = jnp.maximum(
        jnp.dot(x, w1_ref[...], preferred_element_type=jnp.float32)
        + b1_ref[...], 0.0)
    h = jnp.maximum(
        jnp.dot(h, w2_ref[...], preferred_element_type=jnp.float32)
        + b2_ref[...], 0.0)
    o_ref[...] = (
        jnp.dot(h, w3_ref[...], preferred_element_type=jnp.float32)
        + b3_ref[...])


def _mlp(pooled, W1, b1, W2, b2, W3, b3):
    B, D = pooled.shape
    H1 = W1.shape[0]
    NC = W3.shape[0]
    BLK = 1024
    grid = (B // BLK,)
    out = pl.pallas_call(
        _mlp_body,
        grid=grid,
        in_specs=[
            pl.BlockSpec((BLK, D), lambda i: (i, 0)),
            pl.BlockSpec((D, H1), lambda i: (0, 0)),
            pl.BlockSpec((1, H1), lambda i: (0, 0)),
            pl.BlockSpec((H1, D), lambda i: (0, 0)),
            pl.BlockSpec((1, D), lambda i: (0, 0)),
            pl.BlockSpec((D, NC), lambda i: (0, 0)),
            pl.BlockSpec((1, NC), lambda i: (0, 0)),
        ],
        out_specs=pl.BlockSpec((BLK, NC), lambda i: (i, 0)),
        out_shape=jax.ShapeDtypeStruct((B, NC), jnp.float32),
    )(pooled, W1.T, b1[None, :], W2.T, b2[None, :], W3.T, b3[None, :])
    return out


def kernel(text, offsets, table, W1, b1, W2, b2, W3, b3):
    N = text.shape[0]
    B = offsets.shape[0]
    off_ext = jnp.concatenate([offsets, jnp.full((OFF_PAD,), N, jnp.int32)])
    text_pad = jnp.concatenate([text, jnp.zeros((TEXT_PAD,), jnp.int32)])
    pooled = _build_pooled(B, table.shape[1])(text_pad, off_ext, table)
    return _mlp(pooled, W1, b1, W2, b2, W3, b3)


# speedup vs baseline: 1.8680x; 1.0059x over previous
import functools

import jax
import jax.numpy as jnp
from jax import lax
from jax.experimental import pallas as pl
from jax.experimental.pallas import tpu as pltpu
from jax.experimental.pallas import tpu_sc as plsc

NCORES = 2
NSUB = 16
NW = NCORES * NSUB
CK = 128
NBUF = 4
DJ = 8
NWIN = 9
TEXT_PAD = (2 * NBUF + 4) * CK
OFF_PAD = 160


def _build_pooled(B, D):
    bags_per_w = B // NW
    totb = bags_per_w + 1
    mesh = plsc.VectorSubcoreMesh(
        core_axis_name="c", subcore_axis_name="s",
        num_cores=NCORES, num_subcores=NSUB)

    off_len = bags_per_w + OFF_PAD

    @functools.partial(
        pl.kernel,
        out_type=jax.ShapeDtypeStruct((B, D), jnp.float32),
        mesh=mesh,
        scratch_types=[
            pltpu.VMEM((off_len,), jnp.int32),
            pltpu.VMEM((2 * NBUF * CK,), jnp.int32),
            pltpu.VMEM((NBUF, CK, D), jnp.float32),
            pltpu.VMEM((bags_per_w, D), jnp.float32),
            [pltpu.SemaphoreType.DMA] * NBUF,
            pltpu.SemaphoreType.DMA,
        ],
    )
    def pooled_kernel(text_hbm, off_hbm, table_hbm, out_hbm,
                      off_v, idx_v, rows_v, out_v, sems, fsem):
        wid = lax.axis_index("s") * NCORES + lax.axis_index("c")
        base_bag = pl.multiple_of(wid * bags_per_w, 8)
        pltpu.sync_copy(off_hbm.at[pl.ds(base_bag, off_len)], off_v)

        S = off_v[pl.ds(0, 16)][0]
        E = off_v[pl.ds(bags_per_w, 16)][0]
        SA = pl.multiple_of(S & jnp.int32(~(CK - 1)), 8)
        ngrp = jnp.maximum((E - SA + NBUF * CK) // (NBUF * CK),
                           jnp.int32(1))

        def fetch_idx_start(g, p):
            t0 = pl.multiple_of(SA + g * (NBUF * CK), 8)
            po = pl.multiple_of(p * (NBUF * CK), 8)
            pltpu.make_async_copy(
                text_hbm.at[pl.ds(t0, NBUF * CK)],
                idx_v.at[pl.ds(po, NBUF * CK)], fsem).start()

        def fetch_idx_wait(g, p):
            t0 = pl.multiple_of(SA + g * (NBUF * CK), 8)
            po = pl.multiple_of(p * (NBUF * CK), 8)
            pltpu.make_async_copy(
                text_hbm.at[pl.ds(t0, NBUF * CK)],
                idx_v.at[pl.ds(po, NBUF * CK)], fsem).wait()

        def start_gather(s, p):
            po = pl.multiple_of(p * (NBUF * CK) + s * CK, 8)
            pltpu.make_async_copy(
                table_hbm.at[idx_v.at[pl.ds(po, CK)]],
                rows_v.at[s], sems[s]).start()

        def wait_gather(s, p):
            po = pl.multiple_of(p * (NBUF * CK) + s * CK, 8)
            pltpu.make_async_copy(
                table_hbm.at[idx_v.at[pl.ds(po, CK)]],
                rows_v.at[s], sems[s]).wait()

        def row_accum(s, lo, hi, accs):
            def row_body(r, accs):
                return tuple(
                    accs[j] + rows_v[s, r, pl.ds(16 * j, 16)]
                    for j in range(DJ))
            return lax.fori_loop(lo, hi, row_body, accs)

        def process_chunk(c, s, carry):
            bq, accs = carry
            t0 = SA + c * CK
            t_end = t0 + CK
            ones = jnp.ones((16,), jnp.int32)
            zeros = jnp.zeros((16,), jnp.int32)
            cnt_v = zeros
            for w in range(NWIN):
                wv = off_v[pl.ds(bq + 16 * w, 16)]
                cnt_v = cnt_v + jnp.where(wv < t_end, ones, zeros)
            nb = jnp.int32(0)
            for i in range(16):
                nb = nb + cnt_v[i]
            nb = jnp.minimum(nb, totb - bq)

            def bound_body(k, car):
                bqk, pp, accs = car
                pos = off_v[pl.ds(bqk, 16)][0] - t0
                accs = row_accum(s, pp, pos, accs)
                bag = jnp.maximum(bqk - 1, 0)
                for j in range(DJ):
                    out_v[bag, pl.ds(16 * j, 16)] = accs[j]
                z = tuple(jnp.zeros((16,), jnp.float32) for _ in range(DJ))
                return (bqk + 1, pos, z)

            bq, pp, accs = lax.fori_loop(
                0, nb, bound_body, (bq, jnp.int32(0), accs))
            accs = row_accum(s, pp, jnp.int32(CK), accs)
            return (bq, accs)

        fetch_idx_start(jnp.int32(0), 0)
        fetch_idx_wait(jnp.int32(0), 0)
        for s in range(NBUF):
            start_gather(s, 0)

        def group_body(g, carry):
            p = g & 1
            pn = 1 - p
            fetch_idx_start(g + 1, pn)
            fetch_idx_wait(g + 1, pn)
            for s in range(NBUF):
                wait_gather(s, p)
                carry = process_chunk(g * NBUF + s, s, carry)
                start_gather(s, pn)
            return carry

        accs0 = tuple(jnp.zeros((16,), jnp.float32) for _ in range(DJ))
        carry = lax.fori_loop(0, ngrp, group_body, (jnp.int32(0), accs0))

        p_drain = ngrp & 1
        for s in range(NBUF):
            wait_gather(s, p_drain)

        def scale_body(g, _):
            g16 = pl.multiple_of(g * 16, 8)
            lo = off_v[pl.ds(g16, 16)]
            hi = off_v[pl.ds(g16 + 1, 16)]
            cv = jnp.maximum((hi - lo).astype(jnp.float32), jnp.float32(1.0))
            rv = lax.bitcast_convert_type(
                jnp.int32(0x7EF311C3) - lax.bitcast_convert_type(
                    cv, jnp.int32),
                jnp.float32)
            for _ in range(3):
                rv = rv * (jnp.float32(2.0) - cv * rv)
            for jj in range(16):
                b = g16 + jj
                sv = lax.broadcast_in_dim(rv[jj], (16,), ())
                for j in range(DJ):
                    out_v[b, pl.ds(16 * j, 16)] = (
                        out_v[b, pl.ds(16 * j, 16)] * sv)
            return _
        lax.fori_loop(0, bags_per_w // 16, scale_body, 0)

        pltpu.sync_copy(out_v, out_hbm.at[pl.ds(base_bag, bags_per_w)])

    return pooled_kernel


def _mlp_body(x_ref, w1_ref, b1_ref, w2_ref, b2_ref, w3_ref, b3_ref, o_ref):
    x = x_ref[...]
    h = jnp.maximum(
        jnp.dot(x, w1_ref[...], preferred_element_type=jnp.float32)
        + b1_ref[...], 0.0)
    h = jnp.maximum(
        jnp.dot(h, w2_ref[...], preferred_element_type=jnp.float32)
        + b2_ref[...], 0.0)
    o_ref[...] = (
        jnp.dot(h, w3_ref[...], preferred_element_type=jnp.float32)
        + b3_ref[...])


def _mlp(pooled, W1, b1, W2, b2, W3, b3):
    B, D = pooled.shape
    H1 = W1.shape[0]
    NC = W3.shape[0]
    BLK = 2048
    grid = (B // BLK,)
    out = pl.pallas_call(
        _mlp_body,
        grid=grid,
        in_specs=[
            pl.BlockSpec((BLK, D), lambda i: (i, 0)),
            pl.BlockSpec((D, H1), lambda i: (0, 0)),
            pl.BlockSpec((1, H1), lambda i: (0, 0)),
            pl.BlockSpec((H1, D), lambda i: (0, 0)),
            pl.BlockSpec((1, D), lambda i: (0, 0)),
            pl.BlockSpec((D, NC), lambda i: (0, 0)),
            pl.BlockSpec((1, NC), lambda i: (0, 0)),
        ],
        out_specs=pl.BlockSpec((BLK, NC), lambda i: (i, 0)),
        out_shape=jax.ShapeDtypeStruct((B, NC), jnp.float32),
    )(pooled, W1.T, b1[None, :], W2.T, b2[None, :], W3.T, b3[None, :])
    return out


def kernel(text, offsets, table, W1, b1, W2, b2, W3, b3):
    N = text.shape[0]
    B = offsets.shape[0]
    off_ext = jnp.concatenate([offsets, jnp.full((OFF_PAD,), N, jnp.int32)])
    text_pad = jnp.concatenate([text, jnp.zeros((TEXT_PAD,), jnp.int32)])
    pooled = _build_pooled(B, table.shape[1])(text_pad, off_ext, table)
    return _mlp(pooled, W1, b1, W2, b2, W3, b3)
